# baseline (speedup 1.0000x reference)
import functools

import jax
import jax.numpy as jnp
from jax import lax
from jax.experimental import pallas as pl
from jax.experimental.pallas import tpu as pltpu
from jax.experimental.pallas import tpu_sc as plsc

N_ROWS = 512
N_CODES = 8192
DIM = 32
ROW_BLK = 128
N_ROW_BLKS = N_ROWS // ROW_BLK
K_SLOTS = 12
T_MARGIN = 3e-5
D_PAD = 128
CODES_PER_WIDE = D_PAD // DIM
N_GATHER = N_ROWS * K_SLOTS


N_HALF = N_CODES // 2
K_HALF = K_SLOTS // 2


def _cand_kernel(xf_ref, e_ref, cand_ref):
    xm = xf_ref[:, :]
    ee = e_ref[:, :]
    g = lax.dot_general(xm, ee, (((1,), (1,)), ((), ())),
                        preferred_element_type=jnp.float32)
    en = lax.dot_general(jnp.ones((1, DIM), jnp.float32), ee * ee,
                         (((1,), (1,)), ((), ())),
                         preferred_element_type=jnp.float32)
    v = en - (g + g)
    m = jnp.min(v, axis=1, keepdims=True)
    iota = jax.lax.broadcasted_iota(jnp.int32, (ROW_BLK, N_CODES), 1)
    mi = jnp.where(v <= m + T_MARGIN, iota, N_CODES)
    slots = []
    for h in range(2):
        mih = mi[:, h * N_HALF:(h + 1) * N_HALF]
        prev = None
        for _ in range(K_HALF):
            cur_arr = mih if prev is None else jnp.where(mih > prev, mih, N_CODES)
            cur = jnp.min(cur_arr, axis=1, keepdims=True)
            slots.append(cur)
            prev = cur
    cand = jnp.concatenate(slots, axis=1)
    first = jnp.minimum(slots[0], slots[K_HALF])
    cand_ref[:, :] = jnp.where(cand == N_CODES, first, cand)


def _tc_candidates(xf, e):
    return pl.pallas_call(
        _cand_kernel,
        grid=(N_ROW_BLKS,),
        in_specs=[
            pl.BlockSpec((ROW_BLK, DIM), lambda s: (s, 0)),
            pl.BlockSpec((N_CODES, DIM), lambda s: (0, 0)),
        ],
        out_specs=pl.BlockSpec((ROW_BLK, K_SLOTS), lambda s: (s, 0)),
        out_shape=jax.ShapeDtypeStruct((N_ROWS, K_SLOTS), jnp.int32),
    )(xf, e)


def _make_sc_gather():
    info = plsc.get_sparse_core_info()
    nw = info.num_cores * info.num_subcores
    b_per_w = N_GATHER // nw
    mesh = plsc.VectorSubcoreMesh(core_axis_name="c", subcore_axis_name="s")

    @functools.partial(
        pl.kernel, mesh=mesh,
        out_type=jax.ShapeDtypeStruct((N_GATHER, D_PAD), jnp.float32),
        scratch_types=[
            pltpu.VMEM((b_per_w,), jnp.int32),
            pltpu.VMEM((b_per_w, D_PAD), jnp.float32),
            pltpu.SemaphoreType.DMA,
        ],
    )
    def sc_gather(table_hbm, idx_hbm, out_hbm, idx_v, rows_v, sem):
        wid = lax.axis_index("s") * info.num_cores + lax.axis_index("c")
        base = wid * b_per_w
        pltpu.sync_copy(idx_hbm.at[pl.ds(base, b_per_w)], idx_v)
        pltpu.async_copy(table_hbm.at[idx_v], rows_v, sem).wait()
        pltpu.sync_copy(rows_v, out_hbm.at[pl.ds(base, b_per_w)])

    return sc_gather


_sc_gather = _make_sc_gather()


def _exact_distance(xm, ek):
    dd = xm - ek
    t = dd * dd
    u1 = t + pltpu.roll(t, 28, 1)
    u2 = u1 + pltpu.roll(u1, 30, 1)
    u3 = u2 + pltpu.roll(u2, 31, 1)
    d1 = u3 + pltpu.roll(u3, 24, 1)
    d2 = d1 + pltpu.roll(u3, 16, 1)
    d3 = d2 + pltpu.roll(u3, 8, 1)
    return d3[:, 0:1]


def _select_kernel(xf_ref, rows_ref, cand_ref, out_ref):
    xm = xf_ref[:, :]
    cand = cand_ref[:, :]
    dbest = None
    rbest = None
    for k in range(K_SLOTS):
        w = rows_ref[N_ROWS * k:N_ROWS * (k + 1), :]
        off = jnp.bitwise_and(cand[:, k:k + 1], CODES_PER_WIDE - 1)
        offb = jnp.broadcast_to(off, (N_ROWS, DIM))
        ek = None
        for j in range(CODES_PER_WIDE):
            sel = (offb == j).astype(jnp.float32)
            part = sel * w[:, DIM * j:DIM * (j + 1)]
            ek = part if ek is None else ek + part
        d = _exact_distance(xm, ek)
        if dbest is None:
            dbest, rbest = d, ek
        else:
            better = d < dbest
            dbest = jnp.where(better, d, dbest)
            rbest = jnp.where(better, ek, rbest)
    out_ref[:, :] = rbest


def _tc_select(xf, rows, cand):
    return pl.pallas_call(
        _select_kernel,
        in_specs=[
            pl.BlockSpec((N_ROWS, DIM), lambda: (0, 0)),
            pl.BlockSpec((N_GATHER, D_PAD), lambda: (0, 0)),
            pl.BlockSpec((N_ROWS, K_SLOTS), lambda: (0, 0)),
        ],
        out_specs=pl.BlockSpec((N_ROWS, DIM), lambda: (0, 0)),
        out_shape=jax.ShapeDtypeStruct((N_ROWS, DIM), jnp.float32),
    )(xf, rows, cand)


@jax.jit
def kernel(x, embed_weight):
    ori_shape = x.shape
    b, ch, h, w = ori_shape
    xf = jnp.transpose(x, (0, 2, 3, 1)).reshape(b * h * w, ch)
    ew = embed_weight.reshape(N_CODES // CODES_PER_WIDE, D_PAD)

    cand = _tc_candidates(xf, embed_weight)
    widx = (cand.T >> 2).reshape(N_GATHER)
    rows = _sc_gather(ew, widx)
    q = _tc_select(xf, rows, cand)
    return q.reshape(ori_shape)


# speedup vs baseline: 1.0675x; 1.0675x over previous
import functools

import jax
import jax.numpy as jnp
from jax import lax
from jax.experimental import pallas as pl
from jax.experimental.pallas import tpu as pltpu
from jax.experimental.pallas import tpu_sc as plsc

N_ROWS = 512
N_CODES = 8192
DIM = 32
ROW_BLK = 128
N_ROW_BLKS = N_ROWS // ROW_BLK
K_SLOTS = 12
T_MARGIN = 3e-5
D_PAD = 128
CODES_PER_WIDE = D_PAD // DIM
N_GATHER = N_ROWS * K_SLOTS


N_HALF = N_CODES // 2
K_HALF = K_SLOTS // 2


def _cand_kernel(xf_ref, e_ref, cand_ref):
    xm = xf_ref[:, :]
    ee = e_ref[:, :]
    g = lax.dot_general(xm, ee, (((1,), (1,)), ((), ())),
                        preferred_element_type=jnp.float32)
    en = lax.dot_general(jnp.ones((1, DIM), jnp.float32), ee * ee,
                         (((1,), (1,)), ((), ())),
                         preferred_element_type=jnp.float32)
    v = en - (g + g)
    m = jnp.min(v, axis=1, keepdims=True)
    iota = jax.lax.broadcasted_iota(jnp.int32, (ROW_BLK, N_CODES), 1)
    mi = jnp.where(v <= m + T_MARGIN, iota, N_CODES)
    slots = []
    for h in range(2):
        mih = mi[:, h * N_HALF:(h + 1) * N_HALF]
        prev = None
        for _ in range(K_HALF):
            cur_arr = mih if prev is None else jnp.where(mih > prev, mih, N_CODES)
            cur = jnp.min(cur_arr, axis=1, keepdims=True)
            slots.append(cur)
            prev = cur
    cand = jnp.concatenate(slots, axis=1)
    first = jnp.minimum(slots[0], slots[K_HALF])
    cand_ref[:, :] = jnp.where(cand == N_CODES, first, cand)


def _tc_candidates(xf, e):
    return pl.pallas_call(
        _cand_kernel,
        grid=(N_ROW_BLKS,),
        in_specs=[
            pl.BlockSpec((ROW_BLK, DIM), lambda s: (s, 0)),
            pl.BlockSpec((N_CODES, DIM), lambda s: (0, 0)),
        ],
        out_specs=pl.BlockSpec((ROW_BLK, K_SLOTS), lambda s: (s, 0)),
        out_shape=jax.ShapeDtypeStruct((N_ROWS, K_SLOTS), jnp.int32),
    )(xf, e)


def _make_sc_gather():
    info = plsc.get_sparse_core_info()
    nw = info.num_cores * info.num_subcores
    b_per_w = N_GATHER // nw
    mesh = plsc.VectorSubcoreMesh(core_axis_name="c", subcore_axis_name="s")

    @functools.partial(
        pl.kernel, mesh=mesh,
        out_type=jax.ShapeDtypeStruct((N_GATHER, D_PAD), jnp.float32),
        scratch_types=[
            pltpu.VMEM((b_per_w,), jnp.int32),
            pltpu.VMEM((b_per_w, D_PAD), jnp.float32),
            pltpu.SemaphoreType.DMA,
        ],
    )
    def sc_gather(table_hbm, idx_hbm, out_hbm, idx_v, rows_v, sem):
        wid = lax.axis_index("s") * info.num_cores + lax.axis_index("c")
        base = wid * b_per_w
        pltpu.sync_copy(idx_hbm.at[pl.ds(base, b_per_w)], idx_v)
        pltpu.async_copy(table_hbm.at[idx_v], rows_v, sem).wait()
        pltpu.sync_copy(rows_v, out_hbm.at[pl.ds(base, b_per_w)])

    return sc_gather


_sc_gather = _make_sc_gather()


def _exact_distance(xm, ek):
    dd = xm - ek
    t = dd * dd
    d = None
    for r in range(4):
        base = 8 * r
        b = t[:, base:base + 4] + t[:, base + 4:base + 8]
        c = b[:, 0:2] + b[:, 2:4]
        s = c[:, 0:1] + c[:, 1:2]
        d = s if d is None else d + s
    return d


def _select_kernel(xf_ref, rows_ref, cand_ref, out_ref):
    xm = xf_ref[:, :]
    cand = cand_ref[:, :]
    dbest = None
    rbest = None
    for k in range(K_SLOTS):
        w = rows_ref[N_ROWS * k:N_ROWS * (k + 1), :]
        off = jnp.bitwise_and(cand[:, k:k + 1], CODES_PER_WIDE - 1)
        ek = None
        for j in range(CODES_PER_WIDE):
            sel = (off == j).astype(jnp.float32)
            part = sel * w[:, DIM * j:DIM * (j + 1)]
            ek = part if ek is None else ek + part
        d = _exact_distance(xm, ek)
        if dbest is None:
            dbest, rbest = d, ek
        else:
            better = d < dbest
            dbest = jnp.where(better, d, dbest)
            rbest = jnp.where(better, ek, rbest)
    out_ref[:, :] = rbest


def _tc_select(xf, rows, cand):
    return pl.pallas_call(
        _select_kernel,
        in_specs=[
            pl.BlockSpec((N_ROWS, DIM), lambda: (0, 0)),
            pl.BlockSpec((N_GATHER, D_PAD), lambda: (0, 0)),
            pl.BlockSpec((N_ROWS, K_SLOTS), lambda: (0, 0)),
        ],
        out_specs=pl.BlockSpec((N_ROWS, DIM), lambda: (0, 0)),
        out_shape=jax.ShapeDtypeStruct((N_ROWS, DIM), jnp.float32),
    )(xf, rows, cand)


@jax.jit
def kernel(x, embed_weight):
    ori_shape = x.shape
    b, ch, h, w = ori_shape
    xf = jnp.transpose(x, (0, 2, 3, 1)).reshape(b * h * w, ch)
    ew = embed_weight.reshape(N_CODES // CODES_PER_WIDE, D_PAD)

    cand = _tc_candidates(xf, embed_weight)
    widx = (cand.T >> 2).reshape(N_GATHER)
    rows = _sc_gather(ew, widx)
    q = _tc_select(xf, rows, cand)
    return q.reshape(ori_shape)
